# Initial kernel scaffold; baseline (speedup 1.0000x reference)
import jax
import jax.numpy as jnp
from jax.experimental import pallas as pl


def kernel(xyz1, xyz2, normal_rebuild, normal_gt):
    raise NotImplementedError("write your pallas kernel here")


# speedup vs baseline: 2.1064x; 2.1064x over previous
import functools

import jax
import jax.numpy as jnp
from jax import lax
from jax.experimental import pallas as pl
from jax.experimental.pallas import tpu as pltpu
from jax.experimental.pallas import tpu_sc as plsc

B = 4
N = 4096
R = 512
NI = N // R
LANES = 16
CHUNK = 1024
STEPS = CHUNK // LANES


def _tc_body(x1_ref, x2t_ref, nr_ref, ng_ref,
             idx1_ref, idx2_ref, nrout_ref, ngout_ref, dsum_ref,
             bm_ref, sq2_ref, cmin_ref):
    b = pl.program_id(0)
    it = pl.program_id(1)

    @pl.when(jnp.logical_and(b == 0, it == 0))
    def _init_sum():
        dsum_ref[0, 0] = 0.0

    @pl.when(it == 0)
    def _per_batch_setup():
        for ref, out in ((nr_ref, nrout_ref), (ng_ref, ngout_ref)):
            nv = ref[0]
            ss = nv[0:1, :] * nv[0:1, :] + nv[1:2, :] * nv[1:2, :] \
                + nv[2:3, :] * nv[2:3, :]
            scale = 1.0 / jnp.maximum(jnp.sqrt(ss), 1e-12)
            out[0] = nv * scale
        x2 = x2t_ref[0]
        sq2_ref[...] = (x2[0:1, :] * x2[0:1, :] + x2[1:2, :] * x2[1:2, :]
                        + x2[2:3, :] * x2[2:3, :])
        bm_ref[...] = x2
        cmin_ref[...] = jnp.full((1, N), jnp.inf, jnp.float32)

    x1 = x1_ref[0]
    sq1 = (x1[:, 0:1] * x1[:, 0:1] + x1[:, 1:2] * x1[:, 1:2]
           + x1[:, 2:3] * x1[:, 2:3])
    cross = lax.dot_general(x1, bm_ref[...], (((1,), (0,)), ((), ())),
                            preferred_element_type=jnp.float32)
    d = sq1 + sq2_ref[...] - 2.0 * cross

    rmin = jnp.min(d, axis=1, keepdims=True)
    li = lax.broadcasted_iota(jnp.int32, (R, N), 1)
    ridx = jnp.min(jnp.where(d == rmin, li, N), axis=1)
    idx1_ref[0, 0, pl.ds(it * R, R)] = ridx
    dsum_ref[0, 0] += jnp.sum(rmin)

    cmin_t = jnp.min(d, axis=0, keepdims=True)
    si = lax.broadcasted_iota(jnp.int32, (R, N), 0)
    cidx_t = jnp.min(jnp.where(d == cmin_t, si, R), axis=0,
                     keepdims=True) + it * R
    prev = cmin_ref[...]
    better = cmin_t < prev
    @pl.when(it == 0)
    def _col_first():
        cmin_ref[...] = cmin_t
        idx2_ref[0] = cidx_t
    @pl.when(it > 0)
    def _col_merge():
        cmin_ref[...] = jnp.where(better, cmin_t, prev)
        idx2_ref[0] = jnp.where(better, cidx_t, idx2_ref[0])

    @pl.when(it == NI - 1)
    def _finish_batch():
        dsum_ref[0, 0] += jnp.sum(cmin_ref[...])


def _tc_stage(x1, x2t, nr_t, ng_t):
    grid = (B, NI)
    out_shapes = (
        jax.ShapeDtypeStruct((B, 1, N), jnp.int32),
        jax.ShapeDtypeStruct((B, 1, N), jnp.int32),
        jax.ShapeDtypeStruct((B, 3, N), jnp.float32),
        jax.ShapeDtypeStruct((B, 3, N), jnp.float32),
        jax.ShapeDtypeStruct((1, 1), jnp.float32),
    )
    in_specs = [
        pl.BlockSpec((1, R, 3), lambda b, it: (b, it, 0)),
        pl.BlockSpec((1, 3, N), lambda b, it: (b, 0, 0)),
        pl.BlockSpec((1, 3, N), lambda b, it: (b, 0, 0)),
        pl.BlockSpec((1, 3, N), lambda b, it: (b, 0, 0)),
    ]
    out_specs = (
        pl.BlockSpec((1, 1, N), lambda b, it: (b, 0, 0)),
        pl.BlockSpec((1, 1, N), lambda b, it: (b, 0, 0)),
        pl.BlockSpec((1, 3, N), lambda b, it: (b, 0, 0)),
        pl.BlockSpec((1, 3, N), lambda b, it: (b, 0, 0)),
        pl.BlockSpec(memory_space=pltpu.SMEM),
    )
    return pl.pallas_call(
        _tc_body,
        grid=grid,
        in_specs=in_specs,
        out_specs=out_specs,
        out_shape=out_shapes,
        scratch_shapes=[
            pltpu.VMEM((3, N), jnp.float32),
            pltpu.VMEM((1, N), jnp.float32),
            pltpu.VMEM((1, N), jnp.float32),
        ],
    )(x1, x2t, nr_t, ng_t)


def _sc_body(nstack, idxstack, out_hbm, qtab, ttab, idxv, accv):
    info = plsc.get_sparse_core_info()
    nc = info.num_cores
    c = lax.axis_index("c")
    s = lax.axis_index("s")
    wid = s * nc + c
    b = wid // 8
    rem = wid % 8
    dirn = rem // 4
    chunk = rem % 4

    qbase = (b * 2 + dirn) * 3 * N
    tbase = (b * 2 + (1 - dirn)) * 3 * N
    for comp in range(3):
        pltpu.sync_copy(
            nstack.at[pl.ds(qbase + comp * N + chunk * CHUNK, CHUNK)],
            qtab.at[pl.ds(comp * CHUNK, CHUNK)])
        pltpu.sync_copy(nstack.at[pl.ds(tbase + comp * N, N)],
                        ttab.at[pl.ds(comp * N, N)])
    pltpu.sync_copy(
        idxstack.at[pl.ds((b * 2 + dirn) * N + chunk * CHUNK, CHUNK)], idxv)

    def step(k, acc):
        iv = idxv[pl.ds(k * LANES, LANES)]
        sm = jnp.zeros((LANES,), jnp.float32)
        sp = jnp.zeros((LANES,), jnp.float32)
        for comp in range(3):
            q = qtab[pl.ds(comp * CHUNK + k * LANES, LANES)]
            t = plsc.load_gather(ttab, [iv + comp * N])
            sm = sm + jnp.abs(q - t)
            sp = sp + jnp.abs(q + t)
        return acc + jnp.minimum(sm, sp)

    acc = lax.fori_loop(0, STEPS, step, jnp.zeros((LANES,), jnp.float32))
    accv[...] = acc
    pltpu.sync_copy(accv, out_hbm.at[pl.ds(wid * LANES, LANES)])


def _sc_stage(nstack, idxstack):
    mesh = plsc.VectorSubcoreMesh(core_axis_name="c", subcore_axis_name="s")
    f = functools.partial(
        pl.kernel,
        mesh=mesh,
        out_type=jax.ShapeDtypeStruct((32 * LANES,), jnp.float32),
        compiler_params=pltpu.CompilerParams(needs_layout_passes=False),
        scratch_types=[
            pltpu.VMEM((3 * CHUNK,), jnp.float32),
            pltpu.VMEM((3 * N,), jnp.float32),
            pltpu.VMEM((CHUNK,), jnp.int32),
            pltpu.VMEM((LANES,), jnp.float32),
        ],
    )(_sc_body)
    return f(nstack, idxstack)


def kernel(xyz1, xyz2, normal_rebuild, normal_gt):
    x2t = xyz2.transpose(0, 2, 1)
    nr_t = normal_rebuild.transpose(0, 2, 1)
    ng_t = normal_gt.transpose(0, 2, 1)
    idx1, idx2, nr_n, ng_n, dsum = _tc_stage(xyz1, x2t, nr_t, ng_t)
    nstack = jnp.stack([nr_n, ng_n], axis=1).reshape(-1)
    idxstack = jnp.concatenate([idx1, idx2], axis=1).reshape(-1)
    partials = _sc_stage(nstack, idxstack)
    denom = jnp.float32(B * N)
    loss_xyz = dsum[0, 0] / denom
    loss_normal = jnp.sum(partials) / denom
    return (loss_xyz, loss_normal)


# speedup vs baseline: 2.3987x; 1.1388x over previous
import functools

import jax
import jax.numpy as jnp
from jax import lax
from jax.experimental import pallas as pl
from jax.experimental.pallas import tpu as pltpu
from jax.experimental.pallas import tpu_sc as plsc

B = 4
N = 4096
R = 512
NI = N // R
LANES = 16
CHUNK = 1024
STEPS = CHUNK // LANES


def _tc_body(x1_ref, x2t_ref, nr_ref, ng_ref,
             idx1_ref, idx2_ref, nrout_ref, ngout_ref, dsum_ref,
             bm_ref, sq2_ref, cmin_ref):
    b = pl.program_id(0)
    it = pl.program_id(1)

    @pl.when(jnp.logical_and(b == 0, it == 0))
    def _init_sum():
        dsum_ref[0, 0] = 0.0

    @pl.when(it == 0)
    def _per_batch_setup():
        for ref, out in ((nr_ref, nrout_ref), (ng_ref, ngout_ref)):
            nv = ref[0]
            ss = nv[0:1, :] * nv[0:1, :] + nv[1:2, :] * nv[1:2, :] \
                + nv[2:3, :] * nv[2:3, :]
            scale = 1.0 / jnp.maximum(jnp.sqrt(ss), 1e-12)
            out[0] = nv * scale
        x2 = x2t_ref[0]
        sq2_ref[...] = (x2[0:1, :] * x2[0:1, :] + x2[1:2, :] * x2[1:2, :]
                        + x2[2:3, :] * x2[2:3, :])
        bm_ref[...] = x2
        cmin_ref[...] = jnp.full((1, N), jnp.inf, jnp.float32)

    x1 = x1_ref[0]
    sq1 = (x1[:, 0:1] * x1[:, 0:1] + x1[:, 1:2] * x1[:, 1:2]
           + x1[:, 2:3] * x1[:, 2:3])
    cross = lax.dot_general(x1, bm_ref[...], (((1,), (0,)), ((), ())),
                            preferred_element_type=jnp.float32)
    d = sq1 + sq2_ref[...] - 2.0 * cross

    rmin = jnp.min(d, axis=1, keepdims=True)
    li = lax.broadcasted_iota(jnp.int32, (R, N), 1).astype(jnp.float32)
    ridx = jnp.min(jnp.where(d == rmin, li, jnp.float32(N)),
                   axis=1).astype(jnp.int32)
    idx1_ref[0, 0, pl.ds(it * R, R)] = ridx
    dsum_ref[0, 0] += jnp.sum(rmin)

    cmin_t = jnp.min(d, axis=0, keepdims=True)
    si = lax.broadcasted_iota(jnp.int32, (R, N), 0).astype(jnp.float32)
    cidx_t = jnp.min(jnp.where(d == cmin_t, si, jnp.float32(R)), axis=0,
                     keepdims=True).astype(jnp.int32) + it * R
    prev = cmin_ref[...]
    better = cmin_t < prev
    @pl.when(it == 0)
    def _col_first():
        cmin_ref[...] = cmin_t
        idx2_ref[0] = cidx_t
    @pl.when(it > 0)
    def _col_merge():
        cmin_ref[...] = jnp.where(better, cmin_t, prev)
        idx2_ref[0] = jnp.where(better, cidx_t, idx2_ref[0])

    @pl.when(it == NI - 1)
    def _finish_batch():
        dsum_ref[0, 0] += jnp.sum(cmin_ref[...])


def _tc_stage(x1, x2t, nr_t, ng_t):
    grid = (B, NI)
    out_shapes = (
        jax.ShapeDtypeStruct((B, 1, N), jnp.int32),
        jax.ShapeDtypeStruct((B, 1, N), jnp.int32),
        jax.ShapeDtypeStruct((B, 3, N), jnp.float32),
        jax.ShapeDtypeStruct((B, 3, N), jnp.float32),
        jax.ShapeDtypeStruct((1, 1), jnp.float32),
    )
    in_specs = [
        pl.BlockSpec((1, R, 3), lambda b, it: (b, it, 0)),
        pl.BlockSpec((1, 3, N), lambda b, it: (b, 0, 0)),
        pl.BlockSpec((1, 3, N), lambda b, it: (b, 0, 0)),
        pl.BlockSpec((1, 3, N), lambda b, it: (b, 0, 0)),
    ]
    out_specs = (
        pl.BlockSpec((1, 1, N), lambda b, it: (b, 0, 0)),
        pl.BlockSpec((1, 1, N), lambda b, it: (b, 0, 0)),
        pl.BlockSpec((1, 3, N), lambda b, it: (b, 0, 0)),
        pl.BlockSpec((1, 3, N), lambda b, it: (b, 0, 0)),
        pl.BlockSpec(memory_space=pltpu.SMEM),
    )
    return pl.pallas_call(
        _tc_body,
        grid=grid,
        in_specs=in_specs,
        out_specs=out_specs,
        out_shape=out_shapes,
        scratch_shapes=[
            pltpu.VMEM((3, N), jnp.float32),
            pltpu.VMEM((1, N), jnp.float32),
            pltpu.VMEM((1, N), jnp.float32),
        ],
    )(x1, x2t, nr_t, ng_t)


def _sc_body(nstack, idxstack, out_hbm, qtab, ttab, idxv, accv):
    info = plsc.get_sparse_core_info()
    nc = info.num_cores
    c = lax.axis_index("c")
    s = lax.axis_index("s")
    wid = s * nc + c
    b = wid // 8
    rem = wid % 8
    dirn = rem // 4
    chunk = rem % 4

    qbase = (b * 2 + dirn) * 3 * N
    tbase = (b * 2 + (1 - dirn)) * 3 * N
    for comp in range(3):
        pltpu.sync_copy(
            nstack.at[pl.ds(qbase + comp * N + chunk * CHUNK, CHUNK)],
            qtab.at[pl.ds(comp * CHUNK, CHUNK)])
        pltpu.sync_copy(nstack.at[pl.ds(tbase + comp * N, N)],
                        ttab.at[pl.ds(comp * N, N)])
    pltpu.sync_copy(
        idxstack.at[pl.ds((b * 2 + dirn) * N + chunk * CHUNK, CHUNK)], idxv)

    def step(k, acc):
        iv = jnp.minimum(idxv[pl.ds(k * LANES, LANES)], N - 1)
        sm = jnp.zeros((LANES,), jnp.float32)
        sp = jnp.zeros((LANES,), jnp.float32)
        for comp in range(3):
            q = qtab[pl.ds(comp * CHUNK + k * LANES, LANES)]
            t = plsc.load_gather(ttab, [iv + comp * N])
            sm = sm + jnp.abs(q - t)
            sp = sp + jnp.abs(q + t)
        return acc + jnp.minimum(sm, sp)

    acc = lax.fori_loop(0, STEPS, step, jnp.zeros((LANES,), jnp.float32))
    accv[...] = acc
    pltpu.sync_copy(accv, out_hbm.at[pl.ds(wid * LANES, LANES)])


def _sc_stage(nstack, idxstack):
    mesh = plsc.VectorSubcoreMesh(core_axis_name="c", subcore_axis_name="s")
    f = functools.partial(
        pl.kernel,
        mesh=mesh,
        out_type=jax.ShapeDtypeStruct((32 * LANES,), jnp.float32),
        compiler_params=pltpu.CompilerParams(needs_layout_passes=False),
        scratch_types=[
            pltpu.VMEM((3 * CHUNK,), jnp.float32),
            pltpu.VMEM((3 * N,), jnp.float32),
            pltpu.VMEM((CHUNK,), jnp.int32),
            pltpu.VMEM((LANES,), jnp.float32),
        ],
    )(_sc_body)
    return f(nstack, idxstack)


def kernel(xyz1, xyz2, normal_rebuild, normal_gt):
    x2t = xyz2.transpose(0, 2, 1)
    nr_t = normal_rebuild.transpose(0, 2, 1)
    ng_t = normal_gt.transpose(0, 2, 1)
    idx1, idx2, nr_n, ng_n, dsum = _tc_stage(xyz1, x2t, nr_t, ng_t)
    nstack = jnp.stack([nr_n, ng_n], axis=1).reshape(-1)
    idxstack = jnp.concatenate([idx1, idx2], axis=1).reshape(-1)
    partials = _sc_stage(nstack, idxstack)
    denom = jnp.float32(B * N)
    loss_xyz = dsum[0, 0] / denom
    loss_normal = jnp.sum(partials) / denom
    return (loss_xyz, loss_normal)
